# baseline (speedup 1.0000x reference)
import functools

import jax
import jax.numpy as jnp
import numpy as np
from jax.experimental import pallas as pl
from jax.experimental.pallas import tpu as pltpu

_T = [float(np.tan((2 * j + 1) * np.pi / 16)) for j in range(4)]

_OFFS = [(0, 1), (-1, 1), (-1, 0), (-1, -1), (0, -1), (1, -1), (1, 0), (1, 1)]


def _shl(a):
    return jnp.concatenate([a[:, 1:], jnp.zeros_like(a[:, :1])], axis=1)


def _shr(a):
    return jnp.concatenate([jnp.zeros_like(a[:, :1]), a[:, :-1]], axis=1)


def _cshift(a, dx):
    if dx == 1:
        return _shl(a)
    if dx == -1:
        return _shr(a)
    return a


def _sup(a):
    return jnp.concatenate([a[1:], jnp.zeros_like(a[:1])], axis=0)


def _sdn(a):
    return jnp.concatenate([jnp.zeros_like(a[:1]), a[:-1]], axis=0)


def _canny_block(x_ref, t_ref, b_ref, ogx_ref, ogy_ref, omag_ref, oori_ref,
                 ote_ref, *, bh, h, w):
    i = pl.program_id(1)
    n_i = pl.num_programs(1)
    third = jnp.float32(1.0 / 3.0)

    def q(a):
        return a.astype(jnp.bfloat16).astype(jnp.float32)

    m_main = (q(x_ref[0, 0]) + q(x_ref[0, 1]) + q(x_ref[0, 2])) * third
    m_top = (q(t_ref[0, 0]) + q(t_ref[0, 1]) + q(t_ref[0, 2])) * third
    m_top = jnp.where(i > 0, m_top, 0.0)
    m_bot = (q(b_ref[0, 0]) + q(b_ref[0, 1]) + q(b_ref[0, 2])) * third
    m_bot = jnp.where(i < n_i - 1, m_bot, 0.0)
    m = jnp.concatenate([m_top, m_main, m_bot], axis=0)

    g0 = i * bh - 8
    mu, md = _sup(m), _sdn(m)
    vsm = 0.5 * md + m + 0.5 * mu
    drow = mu - md
    gx = _shl(vsm) - _shr(vsm)
    gy = 0.5 * _shr(drow) + drow + 0.5 * _shl(drow)
    mag = jnp.sqrt(gx * gx + gy * gy)

    rows = jax.lax.broadcasted_iota(jnp.int32, (bh + 16, w), 0) + g0
    mag = jnp.where((rows >= 0) & (rows < h), mag, 0.0)

    t = gy / gx
    at = jnp.abs(t)
    u = (at > _T[0]).astype(jnp.float32)
    for thr in _T[1:]:
        u = u + (at > thr).astype(jnp.float32)
    kf = jnp.where(t < 0, -u, u)
    ori = kf * 45.0 + 180.0
    p = kf + 4.0
    p = jnp.where(p == 8.0, 0.0, p)

    mq = q(mag)
    mq_up, mq_dn = _sup(mq), _sdn(mq)
    base = {0: mq, 1: mq_up, -1: mq_dn}
    d = [mq - _cshift(base[dy], dx) for dy, dx in _OFFS]
    remove = None
    for pos in range(4):
        neg = pos + 4
        oriented = (p == pos) | (p == neg)
        not_max = jnp.minimum(d[pos], d[neg]) <= 0.0
        rm = oriented & not_max
        remove = rm if remove is None else (remove | rm)
    thin = jnp.where(remove, 0.0, mag)

    te = (jnp.where(thin > 0.5, 0.5, 0.0) + jnp.where(thin > 1.0, 0.5, 0.0))
    cs = _shl(te) + te + _shr(te)
    hs = _sup(cs) + cs + _sdn(cs)
    te_c = te[8:bh + 8]
    strong = te_c == 1.0
    weak_hi = (hs[8:bh + 8] * 1.25 > 1.0) & (te_c == 0.5)
    out_thin = jnp.where(strong | weak_hi, 1.0, 0.0)

    ogx_ref[0, 0] = gx[8:bh + 8]
    ogy_ref[0, 0] = gy[8:bh + 8]
    omag_ref[0, 0] = mag[8:bh + 8]
    oori_ref[0, 0] = ori[8:bh + 8]
    ote_ref[0, 0] = out_thin


def _run(img):
    b, ch, h, w = img.shape
    bh = 128
    n_i = h // bh
    hb = bh // 8

    out_sds = jax.ShapeDtypeStruct((b, 1, h, w), jnp.float32)
    out_spec = pl.BlockSpec((1, 1, bh, w), lambda bi, i: (bi, 0, i, 0))
    grid = (b, n_i)
    fn = functools.partial(_canny_block, bh=bh, h=h, w=w)
    outs = pl.pallas_call(
        fn,
        grid=grid,
        in_specs=[
            pl.BlockSpec((1, ch, bh, w), lambda bi, i: (bi, 0, i, 0)),
            pl.BlockSpec((1, ch, 8, w),
                         lambda bi, i: (bi, 0, jnp.maximum(hb * i - 1, 0), 0)),
            pl.BlockSpec((1, ch, 8, w),
                         lambda bi, i: (bi, 0,
                                        jnp.minimum(hb * (i + 1), h // 8 - 1),
                                        0)),
        ],
        out_specs=[out_spec] * 5,
        out_shape=[out_sds] * 5,
        compiler_params=pltpu.CompilerParams(
            dimension_semantics=("parallel", "arbitrary")),
    )(img, img, img)
    return tuple(outs)


@jax.jit
def kernel(img):
    b = img.shape[0]
    n_dev = min(2, jax.device_count())
    if n_dev > 1 and b % n_dev == 0:
        mesh = jax.make_mesh((n_dev,), ("x",))
        p = jax.sharding.PartitionSpec("x")
        img = jax.reshard(img, jax.sharding.NamedSharding(mesh, p))
        f = jax.shard_map(_run, mesh=mesh, in_specs=p,
                          out_specs=(p,) * 5, check_vma=False)
        return f(img)
    return _run(img)


# speedup vs baseline: 2.6906x; 2.6906x over previous
import functools

import jax
import jax.numpy as jnp
import numpy as np
from jax.experimental import pallas as pl
from jax.experimental.pallas import tpu as pltpu

_T = [float(np.tan((2 * j + 1) * np.pi / 16)) for j in range(4)]

_OFFS = [(0, 1), (-1, 1), (-1, 0), (-1, -1), (0, -1), (1, -1), (1, 0), (1, 1)]


def _shl(a):
    return jnp.concatenate([a[:, 1:], jnp.zeros_like(a[:, :1])], axis=1)


def _shr(a):
    return jnp.concatenate([jnp.zeros_like(a[:, :1]), a[:, :-1]], axis=1)


def _cshift(a, dx):
    if dx == 1:
        return _shl(a)
    if dx == -1:
        return _shr(a)
    return a


def _sup(a):
    return jnp.concatenate([a[1:], jnp.zeros_like(a[:1])], axis=0)


def _sdn(a):
    return jnp.concatenate([jnp.zeros_like(a[:1]), a[:-1]], axis=0)


def _canny_block(x_ref, t_ref, b_ref, ogx_ref, ogy_ref, omag_ref, oori_ref,
                 ote_ref, *, bh, h, w):
    i = pl.program_id(1)
    n_i = pl.num_programs(1)
    third = jnp.float32(1.0 / 3.0)
    wtop = jnp.where(i > 0, third, 0.0)
    wbot = jnp.where(i < n_i - 1, third, 0.0)

    def q(a):
        return a.astype(jnp.bfloat16).astype(jnp.float32)

    m_main = (q(x_ref[0, 0]) + q(x_ref[0, 1]) + q(x_ref[0, 2])) * third
    m_top = (q(t_ref[0, 0]) + q(t_ref[0, 1]) + q(t_ref[0, 2])) * wtop
    m_bot = (q(b_ref[0, 0]) + q(b_ref[0, 1]) + q(b_ref[0, 2])) * wbot
    m = jnp.concatenate([m_top, m_main, m_bot], axis=0)

    mu, md = _sup(m), _sdn(m)
    vsm = (mu + md) * 0.5 + m
    drow = mu - md
    gx = _shl(vsm) - _shr(vsm)
    gy = (_shl(drow) + _shr(drow)) * 0.5 + drow
    mag = jnp.sqrt(gx * gx + gy * gy)

    son = jnp.where(i > 0, 1.0, 0.0)
    sbn = jnp.where(i < n_i - 1, 1.0, 0.0)
    mag = jnp.concatenate(
        [mag[:8] * son, mag[8:bh + 8], mag[bh + 8:] * sbn], axis=0)

    t = gy / gx
    at = jnp.abs(t)
    u = jnp.where(at > _T[0], 1.0, 0.0)
    for thr in _T[1:]:
        u = u + jnp.where(at > thr, 1.0, 0.0)
    kf = jnp.where(t < 0, -u, u)
    ori = kf * 45.0 + 180.0
    ps = kf + 4.0
    ps = jnp.where(ps >= 8.0, ps - 8.0, ps)
    ps = jnp.where(ps >= 4.0, ps - 4.0, ps)

    mq = q(mag)
    mq_up, mq_dn = _sup(mq), _sdn(mq)
    base = {0: mq, 1: mq_up, -1: mq_dn}
    d = [mq - _cshift(base[dy], dx) for dy, dx in _OFFS]
    dp = jnp.where(ps == 0.0, d[0],
                   jnp.where(ps == 1.0, d[1],
                             jnp.where(ps == 2.0, d[2], d[3])))
    dn = jnp.where(ps == 0.0, d[4],
                   jnp.where(ps == 1.0, d[5],
                             jnp.where(ps == 2.0, d[6], d[7])))
    thin = jnp.where(jnp.minimum(dp, dn) > 0.0, mag, 0.0)

    te = (jnp.where(thin > 0.5, 0.5, 0.0) + jnp.where(thin > 1.0, 0.5, 0.0))
    cs = _shl(te) + te + _shr(te)
    hs = _sup(cs) + cs + _sdn(cs)
    te_c = te[8:bh + 8]
    out_thin = jnp.where(
        te_c == 1.0, 1.0,
        jnp.where(te_c == 0.5,
                  jnp.where(hs[8:bh + 8] * 1.25 > 1.0, 1.0, 0.0), 0.0))

    ogx_ref[0, 0] = gx[8:bh + 8]
    ogy_ref[0, 0] = gy[8:bh + 8]
    omag_ref[0, 0] = mag[8:bh + 8]
    oori_ref[0, 0] = ori[8:bh + 8]
    ote_ref[0, 0] = out_thin


def _run(img):
    b, ch, h, w = img.shape
    bh = 128
    n_i = h // bh
    hb = bh // 8

    out_sds = jax.ShapeDtypeStruct((b, 1, h, w), jnp.float32)
    out_spec = pl.BlockSpec((1, 1, bh, w), lambda bi, i: (bi, 0, i, 0))
    grid = (b, n_i)
    fn = functools.partial(_canny_block, bh=bh, h=h, w=w)
    outs = pl.pallas_call(
        fn,
        grid=grid,
        in_specs=[
            pl.BlockSpec((1, ch, bh, w), lambda bi, i: (bi, 0, i, 0)),
            pl.BlockSpec((1, ch, 8, w),
                         lambda bi, i: (bi, 0, jnp.maximum(hb * i - 1, 0), 0)),
            pl.BlockSpec((1, ch, 8, w),
                         lambda bi, i: (bi, 0,
                                        jnp.minimum(hb * (i + 1), h // 8 - 1),
                                        0)),
        ],
        out_specs=[out_spec] * 5,
        out_shape=[out_sds] * 5,
        compiler_params=pltpu.CompilerParams(
            dimension_semantics=("parallel", "arbitrary")),
    )(img, img, img)
    return tuple(outs)


@jax.jit
def kernel(img):
    return _run(img)


# speedup vs baseline: 2.8043x; 1.0423x over previous
import functools

import jax
import jax.numpy as jnp
import numpy as np
from jax.experimental import pallas as pl
from jax.experimental.pallas import tpu as pltpu

_T = [float(np.tan((2 * j + 1) * np.pi / 16)) for j in range(4)]

_OFFS = [(0, 1), (-1, 1), (-1, 0), (-1, -1), (0, -1), (1, -1), (1, 0), (1, 1)]


def _shl(a):
    return jnp.concatenate([a[:, 1:], jnp.zeros_like(a[:, :1])], axis=1)


def _shr(a):
    return jnp.concatenate([jnp.zeros_like(a[:, :1]), a[:, :-1]], axis=1)


def _cshift(a, dx):
    if dx == 1:
        return _shl(a)
    if dx == -1:
        return _shr(a)
    return a


def _sup(a):
    return jnp.concatenate([a[1:], jnp.zeros_like(a[:1])], axis=0)


def _sdn(a):
    return jnp.concatenate([jnp.zeros_like(a[:1]), a[:-1]], axis=0)


def _canny_block(x_ref, t_ref, b_ref, ogx_ref, ogy_ref, omag_ref, oori_ref,
                 ote_ref, *, bh, h, w):
    i = pl.program_id(1)
    n_i = pl.num_programs(1)
    third = jnp.float32(1.0 / 3.0)
    wtop = jnp.where(i > 0, third, 0.0)
    wbot = jnp.where(i < n_i - 1, third, 0.0)

    def q(a):
        return a.astype(jnp.bfloat16).astype(jnp.float32)

    m_main = (q(x_ref[0, 0]) + q(x_ref[0, 1]) + q(x_ref[0, 2])) * third
    m_top = (q(t_ref[0, 0]) + q(t_ref[0, 1]) + q(t_ref[0, 2])) * wtop
    m_bot = (q(b_ref[0, 0]) + q(b_ref[0, 1]) + q(b_ref[0, 2])) * wbot
    m = jnp.concatenate([m_top, m_main, m_bot], axis=0)

    mu, md = _sup(m), _sdn(m)
    vsm = (mu + md) * 0.5 + m
    drow = mu - md
    gx = _shl(vsm) - _shr(vsm)
    gy = (_shl(drow) + _shr(drow)) * 0.5 + drow
    mag = jnp.sqrt(gx * gx + gy * gy)

    son = jnp.where(i > 0, 1.0, 0.0)
    sbn = jnp.where(i < n_i - 1, 1.0, 0.0)
    mag = jnp.concatenate(
        [mag[:8] * son, mag[8:bh + 8], mag[bh + 8:] * sbn], axis=0)

    t = gy / gx
    at = jnp.abs(t)
    u = jnp.where(at > _T[0], 1.0, 0.0)
    for thr in _T[1:]:
        u = u + jnp.where(at > thr, 1.0, 0.0)
    kf = jnp.where(t < 0, -u, u)
    ori = kf * 45.0 + 180.0
    ps = kf + 4.0
    ps = jnp.where(ps >= 8.0, ps - 8.0, ps)
    ps = jnp.where(ps >= 4.0, ps - 4.0, ps)

    mq = q(mag)
    mq_up, mq_dn = _sup(mq), _sdn(mq)
    base = {0: mq, 1: mq_up, -1: mq_dn}
    d = [mq - _cshift(base[dy], dx) for dy, dx in _OFFS]
    dp = jnp.where(ps == 0.0, d[0],
                   jnp.where(ps == 1.0, d[1],
                             jnp.where(ps == 2.0, d[2], d[3])))
    dn = jnp.where(ps == 0.0, d[4],
                   jnp.where(ps == 1.0, d[5],
                             jnp.where(ps == 2.0, d[6], d[7])))
    thin = jnp.where(jnp.minimum(dp, dn) > 0.0, mag, 0.0)

    te = (jnp.where(thin > 0.5, 0.5, 0.0) + jnp.where(thin > 1.0, 0.5, 0.0))
    cs = _shl(te) + te + _shr(te)
    hs = _sup(cs) + cs + _sdn(cs)
    te_c = te[8:bh + 8]
    out_thin = jnp.where(
        te_c == 1.0, 1.0,
        jnp.where(te_c == 0.5,
                  jnp.where(hs[8:bh + 8] * 1.25 > 1.0, 1.0, 0.0), 0.0))

    ogx_ref[0, 0] = gx[8:bh + 8]
    ogy_ref[0, 0] = gy[8:bh + 8]
    omag_ref[0, 0] = mag[8:bh + 8]
    oori_ref[0, 0] = ori[8:bh + 8]
    ote_ref[0, 0] = out_thin


def _run(img):
    b, ch, h, w = img.shape
    bh = 256
    n_i = h // bh
    hb = bh // 8

    out_sds = jax.ShapeDtypeStruct((b, 1, h, w), jnp.float32)
    out_spec = pl.BlockSpec((1, 1, bh, w), lambda bi, i: (bi, 0, i, 0))
    grid = (b, n_i)
    fn = functools.partial(_canny_block, bh=bh, h=h, w=w)
    outs = pl.pallas_call(
        fn,
        grid=grid,
        in_specs=[
            pl.BlockSpec((1, ch, bh, w), lambda bi, i: (bi, 0, i, 0)),
            pl.BlockSpec((1, ch, 8, w),
                         lambda bi, i: (bi, 0, jnp.maximum(hb * i - 1, 0), 0)),
            pl.BlockSpec((1, ch, 8, w),
                         lambda bi, i: (bi, 0,
                                        jnp.minimum(hb * (i + 1), h // 8 - 1),
                                        0)),
        ],
        out_specs=[out_spec] * 5,
        out_shape=[out_sds] * 5,
        compiler_params=pltpu.CompilerParams(
            dimension_semantics=("parallel", "arbitrary")),
    )(img, img, img)
    return tuple(outs)


@jax.jit
def kernel(img):
    return _run(img)
